# baseline (speedup 1.0000x reference)
import functools

import jax
import jax.numpy as jnp
from jax import lax
from jax.experimental import pallas as pl
from jax.experimental.pallas import tpu as pltpu
from jax.experimental.pallas import tpu_sc as plsc


def _gather_sc(table2, idx_half):
    B = idx_half.shape[0]
    _, E2 = table2.shape
    info = plsc.get_sparse_core_info()
    nw = info.num_cores * info.num_subcores
    b_per_w = B // nw
    mesh = plsc.VectorSubcoreMesh(core_axis_name="c", subcore_axis_name="s")

    @functools.partial(
        pl.kernel,
        mesh=mesh,
        out_type=jax.ShapeDtypeStruct((B, E2), jnp.float32),
        scratch_types=[
            pltpu.VMEM((b_per_w,), jnp.int32),
            pltpu.VMEM((b_per_w, E2), jnp.float32),
            pltpu.SemaphoreType.DMA,
        ],
    )
    def gather_kernel(table_hbm, idx_hbm, out_hbm, idx_v, rows_v, sem):
        wid = lax.axis_index("s") * info.num_cores + lax.axis_index("c")
        base = wid * b_per_w
        pltpu.sync_copy(idx_hbm.at[pl.ds(base, b_per_w)], idx_v)
        pltpu.async_copy(table_hbm.at[idx_v], rows_v, sem).wait()
        pltpu.sync_copy(rows_v, out_hbm.at[pl.ds(base, b_per_w)])

    return gather_kernel(table2, idx_half)


def _matmul_body(h2_ref, par_ref, w_ref, o_ref):
    h2 = h2_ref[...]
    E = h2.shape[1] // 2
    par = par_ref[...]
    hidden = jnp.where(par == 0, h2[:, :E], h2[:, E:])
    o_ref[...] = lax.dot_general(
        hidden,
        w_ref[...],
        (((1,), (1,)), ((), ())),
        preferred_element_type=jnp.float32,
    )


def _project(hidden2, parity, expand_w, bb=512, vb=4096):
    B = hidden2.shape[0]
    V, E = expand_w.shape
    grid = (B // bb, pl.cdiv(V, vb))
    return pl.pallas_call(
        _matmul_body,
        grid=grid,
        in_specs=[
            pl.BlockSpec((bb, 2 * E), lambda i, j: (i, 0)),
            pl.BlockSpec((bb, 1), lambda i, j: (i, 0)),
            pl.BlockSpec((vb, E), lambda i, j: (j, 0)),
        ],
        out_specs=pl.BlockSpec((bb, vb), lambda i, j: (i, j)),
        out_shape=jax.ShapeDtypeStruct((B, V), jnp.float32),
    )(hidden2, parity, expand_w)


def kernel(input, embed_table, expand_w):
    V, E = embed_table.shape
    idx = input.astype(jnp.int32)
    table2 = embed_table.reshape(V // 2, 2 * E)
    hidden2 = _gather_sc(table2, idx // 2)
    parity = (idx & 1).reshape(-1, 1)
    return _project(hidden2, parity, expand_w)


# speedup vs baseline: 1.1565x; 1.1565x over previous
import functools

import jax
import jax.numpy as jnp
from jax import lax
from jax.experimental import pallas as pl
from jax.experimental.pallas import tpu as pltpu
from jax.experimental.pallas import tpu_sc as plsc


def _gather_sc(table2, idx_half):
    B = idx_half.shape[0]
    _, E2 = table2.shape
    info = plsc.get_sparse_core_info()
    nw = info.num_cores * info.num_subcores
    b_per_w = B // nw
    mesh = plsc.VectorSubcoreMesh(core_axis_name="c", subcore_axis_name="s")

    @functools.partial(
        pl.kernel,
        mesh=mesh,
        out_type=jax.ShapeDtypeStruct((B, E2), jnp.float32),
        scratch_types=[
            pltpu.VMEM((b_per_w,), jnp.int32),
            pltpu.VMEM((b_per_w, E2), jnp.float32),
            pltpu.SemaphoreType.DMA,
        ],
    )
    def gather_kernel(table_hbm, idx_hbm, out_hbm, idx_v, rows_v, sem):
        wid = lax.axis_index("s") * info.num_cores + lax.axis_index("c")
        base = wid * b_per_w
        pltpu.sync_copy(idx_hbm.at[pl.ds(base, b_per_w)], idx_v)
        pltpu.async_copy(table_hbm.at[idx_v], rows_v, sem).wait()
        pltpu.sync_copy(rows_v, out_hbm.at[pl.ds(base, b_per_w)])

    return gather_kernel(table2, idx_half)


def _project(hidden2, parity, expand_w, bb=1024, vjb=1024, nbuf=6):
    B = hidden2.shape[0]
    V, E = expand_w.shape
    ni = B // bb
    nvj = pl.cdiv(V, vjb)
    tail = V - (nvj - 1) * vjb
    nt = ni * nvj
    nf = nt - ni

    def body(h2_ref, par_ref, w_ref, o_hbm, o2_ref, hbuf, obuf, sems):
        i = pl.program_id(0)
        j = pl.program_id(1)
        t = i * nvj + j
        f = t - lax.div(t + 1, nvj)
        slot = lax.rem(f, nbuf)

        @pl.when(j == 0)
        def _():
            h2 = h2_ref[...]
            par = par_ref[...]
            hbuf[...] = jnp.where(par == 0, h2[:, :E], h2[:, E:])

        res = lax.dot_general(
            hbuf[...],
            w_ref[...],
            (((1,), (1,)), ((), ())),
            preferred_element_type=jnp.float32,
        )

        @pl.when(j < nvj - 1)
        def _():
            @pl.when(f >= nbuf)
            def _():
                pltpu.make_async_copy(
                    obuf.at[0],
                    o_hbm.at[pl.ds(0, bb), pl.ds(0, vjb)],
                    sems.at[slot],
                ).wait()

            obuf[pl.ds(slot, 1)] = res[None]
            row0 = pl.multiple_of(i * bb, 8)
            col0 = pl.multiple_of(j * vjb, 128)
            pltpu.make_async_copy(
                obuf.at[slot],
                o_hbm.at[pl.ds(row0, bb), pl.ds(col0, vjb)],
                sems.at[slot],
            ).start()

        @pl.when(j == nvj - 1)
        def _():
            o2_ref[...] = res[:, :tail]

        @pl.when(t == nt - 1)
        def _():
            for u in range(min(nbuf, nf)):
                pltpu.make_async_copy(
                    obuf.at[0],
                    o_hbm.at[pl.ds(0, bb), pl.ds(0, vjb)],
                    sems.at[(nf - 1 - u) % nbuf],
                ).wait()

    main, tail_out = pl.pallas_call(
        body,
        grid=(ni, nvj),
        in_specs=[
            pl.BlockSpec((bb, 2 * E), lambda i, j: (i, 0)),
            pl.BlockSpec((bb, 1), lambda i, j: (i, 0)),
            pl.BlockSpec((vjb, E), lambda i, j: (j, 0)),
        ],
        out_specs=[
            pl.BlockSpec(memory_space=pl.ANY),
            pl.BlockSpec((bb, tail), lambda i, j: (i, 0)),
        ],
        out_shape=[
            jax.ShapeDtypeStruct((B, V), jnp.float32),
            jax.ShapeDtypeStruct((B, tail), jnp.float32),
        ],
        scratch_shapes=[
            pltpu.VMEM((bb, E), jnp.float32),
            pltpu.VMEM((nbuf, bb, vjb), jnp.float32),
            pltpu.SemaphoreType.DMA((nbuf,)),
        ],
    )(hidden2, parity, expand_w)
    return lax.dynamic_update_slice(main, tail_out, (0, (nvj - 1) * vjb))


def kernel(input, embed_table, expand_w):
    V, E = embed_table.shape
    idx = input.astype(jnp.int32)
    table2 = embed_table.reshape(V // 2, 2 * E)
    hidden2 = _gather_sc(table2, idx // 2)
    parity = (idx & 1).reshape(-1, 1)
    return _project(hidden2, parity, expand_w)
